# baseline (speedup 1.0000x reference)
import functools

import jax
import jax.numpy as jnp
from jax import lax
from jax.experimental import pallas as pl
from jax.experimental.pallas import tpu as pltpu
from jax.experimental.pallas import tpu_sc as plsc

_B, _T, _K, _CD, _DIM = 8, 4096, 8192, 256, 1024
_N = _B * _T
_NW = 32
_G = 4
_BG = _B // _G
_NG = _BG * _T
_BPW = _NG // _NW
_CHUNK = 128
_NCHUNK = _BPW // _CHUNK


def _sc_gather_body(table_hbm, idx_hbm, out_hbm, idx_v, rows0, rows1, sem0, sem1):
    wid = lax.axis_index("s") * 2 + lax.axis_index("c")
    base = wid * _BPW
    pltpu.sync_copy(idx_hbm.at[wid], idx_v)
    bufs = (rows0, rows1)
    sems = (sem0, sem1)
    cp = pltpu.async_copy(table_hbm.at[idx_v.at[0]], bufs[0], sems[0])
    for c in range(_NCHUNK):
        cp.wait()
        if c + 1 < _NCHUNK:
            cp = pltpu.async_copy(
                table_hbm.at[idx_v.at[c + 1]], bufs[(c + 1) % 2], sems[(c + 1) % 2]
            )
        pltpu.sync_copy(bufs[c % 2], out_hbm.at[pl.ds(base + c * _CHUNK, _CHUNK)])


_sc_gather = functools.partial(
    pl.kernel,
    mesh=plsc.VectorSubcoreMesh(core_axis_name="c", subcore_axis_name="s"),
    out_type=jax.ShapeDtypeStruct((_NG, _CD), jnp.float32),
    scratch_types=[
        pltpu.VMEM((_NCHUNK, _CHUNK), jnp.int32),
        pltpu.VMEM((_CHUNK, _CD), jnp.float32),
        pltpu.VMEM((_CHUNK, _CD), jnp.float32),
        pltpu.SemaphoreType.DMA,
        pltpu.SemaphoreType.DMA,
    ],
)(_sc_gather_body)


def _mm_body(x_ref, w_ref, b_ref, o_ref):
    x = x_ref[0]
    w = w_ref[...]
    acc = lax.dot_general(
        w, x, (((1,), (1,)), ((), ())), preferred_element_type=jnp.float32
    )
    o_ref[0] = acc + b_ref[...]


def _mm_call(gathered_g, W, b2):
    return pl.pallas_call(
        _mm_body,
        grid=(_BG,),
        in_specs=[
            pl.BlockSpec((1, _T, _CD), lambda bb: (bb, 0, 0)),
            pl.BlockSpec((_DIM, _CD), lambda bb: (0, 0)),
            pl.BlockSpec((_DIM, 1), lambda bb: (0, 0)),
        ],
        out_specs=pl.BlockSpec((1, _DIM, _T), lambda bb: (bb, 0, 0)),
        out_shape=jax.ShapeDtypeStruct((_BG, _DIM, _T), jnp.float32),
    )(gathered_g.reshape(_BG, _T, _CD), W, b2)


def kernel(indices, codebook, W, b):
    idx = indices.reshape(_G, _NW, _NCHUNK, _CHUNK).astype(jnp.int32)
    b2 = b.reshape(_DIM, 1)
    outs = []
    for g in range(_G):
        gathered = _sc_gather(codebook, idx[g])
        outs.append(_mm_call(gathered, W, b2))
    return jnp.concatenate(outs, axis=0)


# speedup vs baseline: 1.8271x; 1.8271x over previous
import functools

import jax
import jax.numpy as jnp
from jax import lax
from jax.experimental import pallas as pl
from jax.experimental.pallas import tpu as pltpu
from jax.experimental.pallas import tpu_sc as plsc

_B, _T, _K, _CD, _DIM = 8, 4096, 8192, 256, 1024
_CDP = _CD // 2
_N = _B * _T
_NW = 32
_BPW = _N // _NW
_CHUNK = 128
_NCHUNK = _BPW // _CHUNK
_PB = 1024


def _pack_body(u_ref, o_ref):
    u = u_ref[...]
    r = (u + jnp.int32(32767) + ((u >> 16) & jnp.int32(1))) >> 16
    o_ref[...] = (r[:, :_CDP] & jnp.int32(0xFFFF)) | (r[:, _CDP:] << 16)


def _sc_gather_body(table_hbm, idx_hbm, out_hbm, idx_v, rows0, rows1, sem0, sem1):
    wid = lax.axis_index("s") * 2 + lax.axis_index("c")
    base = wid * _BPW
    pltpu.sync_copy(idx_hbm.at[wid], idx_v)
    bufs = (rows0, rows1)
    sems = (sem0, sem1)
    cp = pltpu.async_copy(table_hbm.at[idx_v.at[0]], bufs[0], sems[0])
    for c in range(_NCHUNK):
        cp.wait()
        if c + 1 < _NCHUNK:
            cp = pltpu.async_copy(
                table_hbm.at[idx_v.at[c + 1]], bufs[(c + 1) % 2], sems[(c + 1) % 2]
            )
        pltpu.sync_copy(bufs[c % 2], out_hbm.at[pl.ds(base + c * _CHUNK, _CHUNK)])


_sc_gather = functools.partial(
    pl.kernel,
    mesh=plsc.VectorSubcoreMesh(core_axis_name="c", subcore_axis_name="s"),
    out_type=jax.ShapeDtypeStruct((_N, _CDP), jnp.int32),
    scratch_types=[
        pltpu.VMEM((_NCHUNK, _CHUNK), jnp.int32),
        pltpu.VMEM((_CHUNK, _CDP), jnp.int32),
        pltpu.VMEM((_CHUNK, _CDP), jnp.int32),
        pltpu.SemaphoreType.DMA,
        pltpu.SemaphoreType.DMA,
    ],
)(_sc_gather_body)


def _mm_body(x_ref, wl_ref, wh_ref, b_ref, o_ref):
    x = x_ref[0]
    xl = lax.bitcast_convert_type(x << 16, jnp.float32)
    xh = lax.bitcast_convert_type(x & jnp.int32(-65536), jnp.float32)
    dn = (((1,), (1,)), ((), ()))
    acc = lax.dot_general(wl_ref[...], xl, dn, preferred_element_type=jnp.float32)
    acc += lax.dot_general(wh_ref[...], xh, dn, preferred_element_type=jnp.float32)
    o_ref[0] = acc + b_ref[...]


def kernel(indices, codebook, W, b):
    idx = indices.reshape(_NW, _NCHUNK, _CHUNK).astype(jnp.int32)
    cb_bits = lax.bitcast_convert_type(codebook, jnp.int32)
    packed = pl.pallas_call(
        _pack_body,
        grid=(_K // _PB,),
        in_specs=[pl.BlockSpec((_PB, _CD), lambda i: (i, 0))],
        out_specs=pl.BlockSpec((_PB, _CDP), lambda i: (i, 0)),
        out_shape=jax.ShapeDtypeStruct((_K, _CDP), jnp.int32),
    )(cb_bits)
    gathered = _sc_gather(packed, idx)
    out = pl.pallas_call(
        _mm_body,
        grid=(_B,),
        in_specs=[
            pl.BlockSpec((1, _T, _CDP), lambda bb: (bb, 0, 0)),
            pl.BlockSpec((_DIM, _CDP), lambda bb: (0, 0)),
            pl.BlockSpec((_DIM, _CDP), lambda bb: (0, 0)),
            pl.BlockSpec((_DIM, 1), lambda bb: (0, 0)),
        ],
        out_specs=pl.BlockSpec((1, _DIM, _T), lambda bb: (bb, 0, 0)),
        out_shape=jax.ShapeDtypeStruct((_B, _DIM, _T), jnp.float32),
    )(gathered.reshape(_B, _T, _CDP), W[:, :_CDP], W[:, _CDP:], b.reshape(_DIM, 1))
    return out


# speedup vs baseline: 1.8919x; 1.0355x over previous
import functools

import jax
import jax.numpy as jnp
from jax import lax
from jax.experimental import pallas as pl
from jax.experimental.pallas import tpu as pltpu
from jax.experimental.pallas import tpu_sc as plsc

_B, _T, _K, _CD, _DIM = 8, 4096, 8192, 256, 1024
_CDP = _CD // 2
_N = _B * _T
_NW = 32
_BPW = _N // _NW
_CHUNK = 128
_NCHUNK = _BPW // _CHUNK
_PB = 1024


def _pack_body(u_ref, o_ref):
    u = u_ref[...]
    r = (u + jnp.int32(32767) + ((u >> 16) & jnp.int32(1))) >> 16
    o_ref[...] = (r[:, :_CDP] & jnp.int32(0xFFFF)) | (r[:, _CDP:] << 16)


def _sc_gather_body(table_hbm, idx_hbm, out_hbm, idx_v, rows0, rows1, sem0, sem1):
    wid = lax.axis_index("s") * 2 + lax.axis_index("c")
    base = wid * _BPW
    pltpu.sync_copy(idx_hbm.at[wid], idx_v)
    bufs = (rows0, rows1)
    sems = (sem0, sem1)
    cp = pltpu.async_copy(table_hbm.at[idx_v.at[0]], bufs[0], sems[0])
    for c in range(_NCHUNK):
        cp.wait()
        if c + 1 < _NCHUNK:
            cp = pltpu.async_copy(
                table_hbm.at[idx_v.at[c + 1]], bufs[(c + 1) % 2], sems[(c + 1) % 2]
            )
        pltpu.sync_copy(bufs[c % 2], out_hbm.at[pl.ds(base + c * _CHUNK, _CHUNK)])


_sc_gather = functools.partial(
    pl.kernel,
    mesh=plsc.VectorSubcoreMesh(core_axis_name="c", subcore_axis_name="s"),
    out_type=jax.ShapeDtypeStruct((_N, _CDP), jnp.int32),
    scratch_types=[
        pltpu.VMEM((_NCHUNK, _CHUNK), jnp.int32),
        pltpu.VMEM((_CHUNK, _CDP), jnp.int32),
        pltpu.VMEM((_CHUNK, _CDP), jnp.int32),
        pltpu.SemaphoreType.DMA,
        pltpu.SemaphoreType.DMA,
    ],
)(_sc_gather_body)


def _mm_body(x_ref, w_ref, b_ref, o_ref):
    x = x_ref[0]
    xl = lax.bitcast_convert_type(x << 16, jnp.float32)
    xh = lax.bitcast_convert_type(x & jnp.int32(-65536), jnp.float32)
    x2 = jnp.concatenate([xl, xh], axis=1)
    acc = lax.dot_general(
        w_ref[...], x2, (((1,), (1,)), ((), ())), preferred_element_type=jnp.float32
    )
    o_ref[0] = acc + b_ref[...]


def kernel(indices, codebook, W, b):
    idx = indices.reshape(_NW, _NCHUNK, _CHUNK).astype(jnp.int32)
    cb_bits = lax.bitcast_convert_type(codebook, jnp.int32)
    packed = pl.pallas_call(
        _pack_body,
        grid=(_K // _PB,),
        in_specs=[pl.BlockSpec((_PB, _CD), lambda i: (i, 0))],
        out_specs=pl.BlockSpec((_PB, _CDP), lambda i: (i, 0)),
        out_shape=jax.ShapeDtypeStruct((_K, _CDP), jnp.int32),
    )(cb_bits)
    gathered = _sc_gather(packed, idx)
    out = pl.pallas_call(
        _mm_body,
        grid=(_B,),
        in_specs=[
            pl.BlockSpec((1, _T, _CDP), lambda bb: (bb, 0, 0)),
            pl.BlockSpec((_DIM, _CD), lambda bb: (0, 0)),
            pl.BlockSpec((_DIM, 1), lambda bb: (0, 0)),
        ],
        out_specs=pl.BlockSpec((1, _DIM, _T), lambda bb: (bb, 0, 0)),
        out_shape=jax.ShapeDtypeStruct((_B, _DIM, _T), jnp.float32),
    )(gathered.reshape(_B, _T, _CDP), W, b.reshape(_DIM, 1))
    return out
